# baseline (speedup 1.0000x reference)
import functools

import jax
import jax.numpy as jnp
from jax import lax
from jax.experimental import pallas as pl
from jax.experimental.pallas import tpu as pltpu
from jax.experimental.pallas import tpu_sc as plsc

BATCH = 16384
EMBED_DIM = 32
N_USERS = 1000000
N_MOVIES = 100000
NUM_CORES = 2
NUM_SUBCORES = 16
NUM_WORKERS = NUM_CORES * NUM_SUBCORES
BPW = BATCH // NUM_WORKERS
CHUNK = 128
NCHUNK = BPW // CHUNK
LANES = 16


_TC_BLK = 8192
_TC_NBUF = 8


_TC_NCORE = 2
_TC_OBUF = 4

_U_CHUNKS = 61
_U_TAIL = N_USERS - _TC_NCORE * _U_CHUNKS * _TC_BLK
_M_CHUNKS = 6
_M_TAIL = N_MOVIES - _TC_NCORE * _M_CHUNKS * _TC_BLK


def _fused_colsum_body(u_ref, m_ref, w_ref, ou_ref, om_ref,
                       bufs, obuf, utin, mtin, utout, mtout,
                       sems, osems, tsem):
    c = pl.program_id(0)
    ubase = c * (_U_CHUNKS * _TC_BLK)
    mbase = c * (_M_CHUNKS * _TC_BLK)
    chunks = ([(u_ref, ou_ref, 0, ubase + i * _TC_BLK)
               for i in range(_U_CHUNKS)]
              + [(m_ref, om_ref, 1, mbase + i * _TC_BLK)
                 for i in range(_M_CHUNKS)])
    n = len(chunks)

    def issue(k):
        t_ref, _, _, off = chunks[k]
        return pltpu.make_async_copy(
            t_ref.at[:, pl.ds(off, _TC_BLK)], bufs.at[k % _TC_NBUF],
            sems.at[k % _TC_NBUF])

    for k in range(min(_TC_NBUF, n)):
        issue(k).start()
    wvals = [w_ref[:, 0:1], w_ref[:, 1:2]]
    out_copies = [None] * n
    for k in range(n):
        _, o_ref, wsel, off = chunks[k]
        if k >= _TC_OBUF:
            out_copies[k - _TC_OBUF].wait()
        issue(k).wait()
        obuf[k % _TC_OBUF] = jnp.sum(bufs[k % _TC_NBUF] * wvals[wsel],
                                     axis=0)
        cp = pltpu.make_async_copy(
            obuf.at[k % _TC_OBUF], o_ref.at[pl.ds(off, _TC_BLK)],
            osems.at[k % _TC_OBUF])
        cp.start()
        out_copies[k] = cp
        if k + _TC_NBUF < n:
            issue(k + _TC_NBUF).start()

    @pl.when(c == 1)
    def _():
        off = _TC_NCORE * _U_CHUNKS * _TC_BLK
        cp = pltpu.make_async_copy(u_ref.at[:, pl.ds(off, _U_TAIL)], utin,
                                   tsem)
        cp.start()
        cp.wait()
        utout[...] = jnp.sum(utin[...] * wvals[0], axis=0)
        cp2 = pltpu.make_async_copy(utout, ou_ref.at[pl.ds(off, _U_TAIL)],
                                    tsem)
        cp2.start()
        cp2.wait()

    @pl.when(c == 0)
    def _():
        off = _TC_NCORE * _M_CHUNKS * _TC_BLK
        cp = pltpu.make_async_copy(m_ref.at[:, pl.ds(off, _M_TAIL)], mtin,
                                   tsem)
        cp.start()
        cp.wait()
        mtout[...] = jnp.sum(mtin[...] * wvals[1], axis=0)
        cp2 = pltpu.make_async_copy(mtout, om_ref.at[pl.ds(off, _M_TAIL)],
                                    tsem)
        cp2.start()
        cp2.wait()

    for k in range(max(0, n - _TC_OBUF), n):
        out_copies[k].wait()


def _fused_colsum(user_t, movie_t, w_mat):
    return pl.pallas_call(
        _fused_colsum_body,
        grid=(_TC_NCORE,),
        in_specs=[
            pl.BlockSpec(memory_space=pltpu.MemorySpace.HBM),
            pl.BlockSpec(memory_space=pltpu.MemorySpace.HBM),
            pl.BlockSpec((EMBED_DIM, 2), lambda c: (0, 0)),
        ],
        out_specs=[
            pl.BlockSpec(memory_space=pltpu.MemorySpace.HBM),
            pl.BlockSpec(memory_space=pltpu.MemorySpace.HBM),
        ],
        out_shape=[
            jax.ShapeDtypeStruct((N_USERS,), jnp.float32),
            jax.ShapeDtypeStruct((N_MOVIES,), jnp.float32),
        ],
        scratch_shapes=[
            pltpu.VMEM((_TC_NBUF, EMBED_DIM, _TC_BLK), jnp.float32),
            pltpu.VMEM((_TC_OBUF, _TC_BLK), jnp.float32),
            pltpu.VMEM((EMBED_DIM, _U_TAIL), jnp.float32),
            pltpu.VMEM((EMBED_DIM, _M_TAIL), jnp.float32),
            pltpu.VMEM((_U_TAIL,), jnp.float32),
            pltpu.VMEM((_M_TAIL,), jnp.float32),
            pltpu.SemaphoreType.DMA((_TC_NBUF,)),
            pltpu.SemaphoreType.DMA((_TC_OBUF,)),
            pltpu.SemaphoreType.DMA,
        ],
        compiler_params=pltpu.CompilerParams(
            dimension_semantics=("parallel",)),
    )(user_t, movie_t, w_mat)


_mesh = plsc.VectorSubcoreMesh(
    core_axis_name="c", subcore_axis_name="s", num_cores=NUM_CORES,
    num_subcores=NUM_SUBCORES)


@functools.partial(
    pl.kernel,
    out_type=jax.ShapeDtypeStruct((BATCH,), jnp.float32),
    mesh=_mesh,
    compiler_params=pltpu.CompilerParams(needs_layout_passes=False,
                                         use_tc_tiling_on_sc=False),
    scratch_types=[
        pltpu.VMEM((BPW,), jnp.int32),
        pltpu.VMEM((BPW,), jnp.int32),
        pltpu.VMEM((BPW,), jnp.float32),
        pltpu.VMEM((BPW,), jnp.float32),
        pltpu.VMEM((LANES,), jnp.float32),
        pltpu.VMEM((BPW,), jnp.float32),
        pltpu.SemaphoreType.DMA,
        pltpu.SemaphoreType.DMA,
    ],
)
def _gather_add(users_hbm, movies_hbm, uw_hbm, mw_hbm, b_hbm, out_hbm,
                uidx, midx, gu, gm, bvec, outv, sem_u, sem_m):
    wid = lax.axis_index("s") * NUM_CORES + lax.axis_index("c")
    base = wid * BPW
    pltpu.sync_copy(users_hbm.at[pl.ds(base, BPW)], uidx)
    pltpu.sync_copy(movies_hbm.at[pl.ds(base, BPW)], midx)
    pltpu.sync_copy(b_hbm, bvec)
    copies = []
    for c in range(NCHUNK):
        sl = pl.ds(c * CHUNK, CHUNK)
        copies.append(pltpu.async_copy(uw_hbm.at[uidx.at[sl]], gu.at[sl],
                                       sem_u))
        copies.append(pltpu.async_copy(mw_hbm.at[midx.at[sl]], gm.at[sl],
                                       sem_m))
    for cp in copies:
        cp.wait()
    b_val = bvec[...]
    for s in range(BPW // LANES):
        sl = pl.ds(s * LANES, LANES)
        outv[sl] = gu[sl] + gm[sl] + b_val
    pltpu.sync_copy(outv, out_hbm.at[pl.ds(base, BPW)])


def kernel(users, movies, user_table, movie_table, W, b):
    w = W.reshape(-1)
    w_mat = jnp.stack([w[:EMBED_DIM], w[EMBED_DIM:]], axis=1)
    uw, mw = _fused_colsum(user_table.T, movie_table.T, w_mat)
    bvec = jnp.broadcast_to(b.reshape(()), (LANES,))
    out = _gather_add(users.astype(jnp.int32), movies.astype(jnp.int32),
                      uw, mw, bvec)
    return out.reshape(BATCH, 1)


# speedup vs baseline: 1.0252x; 1.0252x over previous
import functools

import jax
import jax.numpy as jnp
from jax import lax
from jax.experimental import pallas as pl
from jax.experimental.pallas import tpu as pltpu
from jax.experimental.pallas import tpu_sc as plsc

BATCH = 16384
EMBED_DIM = 32
N_USERS = 1000000
N_MOVIES = 100000
NUM_CORES = 2
NUM_SUBCORES = 16
NUM_WORKERS = NUM_CORES * NUM_SUBCORES
BPW = BATCH // NUM_WORKERS
CHUNK = 128
NCHUNK = BPW // CHUNK
LANES = 16


_TC_BLK = 8192
_TC_NBUF = 8


def _colsum_body(n, r, t_ref, w_ref, o_ref, bufs, rb, sems, rs):
    def issue(k):
        slot = k % _TC_NBUF
        return pltpu.make_async_copy(
            t_ref.at[:, pl.ds(k * _TC_BLK, _TC_BLK)], bufs.at[slot],
            sems.at[slot])

    for k in range(min(_TC_NBUF, n)):
        issue(k).start()
    rcopy = pltpu.make_async_copy(
        t_ref.at[:, pl.ds(n * _TC_BLK, r)], rb, rs)
    rcopy.start()
    wval = w_ref[...]
    for k in range(n):
        issue(k).wait()
        o_ref[pl.ds(k * _TC_BLK, _TC_BLK)] = jnp.sum(
            bufs[k % _TC_NBUF] * wval, axis=0)
        if k + _TC_NBUF < n:
            issue(k + _TC_NBUF).start()
    rcopy.wait()
    o_ref[pl.ds(n * _TC_BLK, r)] = jnp.sum(rb[...] * wval, axis=0)


def _colsum(table_t, w_col, size):
    n, r = divmod(size, _TC_BLK)
    return pl.pallas_call(
        functools.partial(_colsum_body, n, r),
        in_specs=[
            pl.BlockSpec(memory_space=pltpu.MemorySpace.HBM),
            pl.BlockSpec((EMBED_DIM, 1), lambda: (0, 0)),
        ],
        out_specs=pl.BlockSpec((size,), lambda: (0,)),
        out_shape=jax.ShapeDtypeStruct((size,), jnp.float32),
        scratch_shapes=[
            pltpu.VMEM((_TC_NBUF, EMBED_DIM, _TC_BLK), jnp.float32),
            pltpu.VMEM((EMBED_DIM, r), jnp.float32),
            pltpu.SemaphoreType.DMA((_TC_NBUF,)),
            pltpu.SemaphoreType.DMA,
        ],
    )(table_t, w_col)


_mesh = plsc.VectorSubcoreMesh(
    core_axis_name="c", subcore_axis_name="s", num_cores=NUM_CORES,
    num_subcores=NUM_SUBCORES)


@functools.partial(
    pl.kernel,
    out_type=jax.ShapeDtypeStruct((BATCH,), jnp.float32),
    mesh=_mesh,
    compiler_params=pltpu.CompilerParams(needs_layout_passes=False,
                                         use_tc_tiling_on_sc=False),
    scratch_types=[
        pltpu.VMEM((BPW,), jnp.int32),
        pltpu.VMEM((BPW,), jnp.float32),
        pltpu.VMEM((LANES,), jnp.float32),
        pltpu.VMEM((BPW,), jnp.float32),
        pltpu.SemaphoreType.DMA,
    ],
)
def _gather_m(movies_hbm, mw_hbm, b_hbm, out_hbm,
              midx, gm, bvec, outv, sem_m):
    wid = lax.axis_index("s") * NUM_CORES + lax.axis_index("c")
    base = wid * BPW
    pltpu.sync_copy(movies_hbm.at[pl.ds(base, BPW)], midx)
    pltpu.sync_copy(b_hbm, bvec)
    copies = []
    for c in range(NCHUNK):
        sl = pl.ds(c * CHUNK, CHUNK)
        copies.append(pltpu.async_copy(mw_hbm.at[midx.at[sl]], gm.at[sl],
                                       sem_m))
    for cp in copies:
        cp.wait()
    b_val = bvec[...]
    for s in range(BPW // LANES):
        sl = pl.ds(s * LANES, LANES)
        outv[sl] = gm[sl] + b_val
    pltpu.sync_copy(outv, out_hbm.at[pl.ds(base, BPW)])


@functools.partial(
    pl.kernel,
    out_type=jax.ShapeDtypeStruct((BATCH,), jnp.float32),
    mesh=_mesh,
    compiler_params=pltpu.CompilerParams(needs_layout_passes=False,
                                         use_tc_tiling_on_sc=False),
    scratch_types=[
        pltpu.VMEM((BPW,), jnp.int32),
        pltpu.VMEM((BPW,), jnp.float32),
        pltpu.VMEM((BPW,), jnp.float32),
        pltpu.VMEM((BPW,), jnp.float32),
        pltpu.SemaphoreType.DMA,
    ],
)
def _gather_u_add(users_hbm, uw_hbm, gmb_hbm, out_hbm,
                  uidx, gu, gmb, outv, sem_u):
    wid = lax.axis_index("s") * NUM_CORES + lax.axis_index("c")
    base = wid * BPW
    pltpu.sync_copy(users_hbm.at[pl.ds(base, BPW)], uidx)
    pltpu.sync_copy(gmb_hbm.at[pl.ds(base, BPW)], gmb)
    copies = []
    for c in range(NCHUNK):
        sl = pl.ds(c * CHUNK, CHUNK)
        copies.append(pltpu.async_copy(uw_hbm.at[uidx.at[sl]], gu.at[sl],
                                       sem_u))
    for cp in copies:
        cp.wait()
    for s in range(BPW // LANES):
        sl = pl.ds(s * LANES, LANES)
        outv[sl] = gu[sl] + gmb[sl]
    pltpu.sync_copy(outv, out_hbm.at[pl.ds(base, BPW)])


def kernel(users, movies, user_table, movie_table, W, b):
    w = W.reshape(-1)
    wu = w[:EMBED_DIM].reshape(EMBED_DIM, 1)
    wm = w[EMBED_DIM:].reshape(EMBED_DIM, 1)
    bvec = jnp.broadcast_to(b.reshape(()), (LANES,))
    mw = _colsum(movie_table.T, wm, N_MOVIES)
    gmb = _gather_m(movies.astype(jnp.int32), mw, bvec)
    uw = _colsum(user_table.T, wu, N_USERS)
    out = _gather_u_add(users.astype(jnp.int32), uw, gmb)
    return out.reshape(BATCH, 1)


# speedup vs baseline: 1.0612x; 1.0351x over previous
import functools

import jax
import jax.numpy as jnp
from jax import lax
from jax.experimental import pallas as pl
from jax.experimental.pallas import tpu as pltpu
from jax.experimental.pallas import tpu_sc as plsc

BATCH = 16384
EMBED_DIM = 32
N_USERS = 1000000
N_MOVIES = 100000
NUM_CORES = 2
NUM_SUBCORES = 16
NUM_WORKERS = NUM_CORES * NUM_SUBCORES
BPW = BATCH // NUM_WORKERS
CHUNK = 128
NCHUNK = BPW // CHUNK
LANES = 16


_TC_BLK = 8192
_TC_NBUF = 8


def _fused_colsum_body(nu, ru, nm, rm, u_ref, m_ref, w_ref,
                       ou_ref, om_ref, bufs, rub, rmb, sems, rus, rms):
    chunks = ([(u_ref, ou_ref, 0, i) for i in range(nu)]
              + [(m_ref, om_ref, 1, i) for i in range(nm)])
    n = len(chunks)

    def issue(k):
        t_ref, _, _, i = chunks[k]
        slot = k % _TC_NBUF
        return pltpu.make_async_copy(
            t_ref.at[:, pl.ds(i * _TC_BLK, _TC_BLK)], bufs.at[slot],
            sems.at[slot])

    for k in range(min(_TC_NBUF, n)):
        issue(k).start()
    rucopy = pltpu.make_async_copy(
        u_ref.at[:, pl.ds(nu * _TC_BLK, ru)], rub, rus)
    rucopy.start()
    rmcopy = pltpu.make_async_copy(
        m_ref.at[:, pl.ds(nm * _TC_BLK, rm)], rmb, rms)
    rmcopy.start()
    wvals = [w_ref[:, 0:1], w_ref[:, 1:2]]
    for k in range(n):
        _, o_ref, wsel, i = chunks[k]
        issue(k).wait()
        o_ref[pl.ds(i * _TC_BLK, _TC_BLK)] = jnp.sum(
            bufs[k % _TC_NBUF] * wvals[wsel], axis=0)
        if k + _TC_NBUF < n:
            issue(k + _TC_NBUF).start()
    rucopy.wait()
    ou_ref[pl.ds(nu * _TC_BLK, ru)] = jnp.sum(rub[...] * wvals[0], axis=0)
    rmcopy.wait()
    om_ref[pl.ds(nm * _TC_BLK, rm)] = jnp.sum(rmb[...] * wvals[1], axis=0)


def _fused_colsum(user_t, movie_t, w_mat):
    nu, ru = divmod(N_USERS, _TC_BLK)
    nm, rm = divmod(N_MOVIES, _TC_BLK)
    return pl.pallas_call(
        functools.partial(_fused_colsum_body, nu, ru, nm, rm),
        in_specs=[
            pl.BlockSpec(memory_space=pltpu.MemorySpace.HBM),
            pl.BlockSpec(memory_space=pltpu.MemorySpace.HBM),
            pl.BlockSpec((EMBED_DIM, 2), lambda: (0, 0)),
        ],
        out_specs=[
            pl.BlockSpec((N_USERS,), lambda: (0,)),
            pl.BlockSpec((N_MOVIES,), lambda: (0,)),
        ],
        out_shape=[
            jax.ShapeDtypeStruct((N_USERS,), jnp.float32),
            jax.ShapeDtypeStruct((N_MOVIES,), jnp.float32),
        ],
        scratch_shapes=[
            pltpu.VMEM((_TC_NBUF, EMBED_DIM, _TC_BLK), jnp.float32),
            pltpu.VMEM((EMBED_DIM, ru), jnp.float32),
            pltpu.VMEM((EMBED_DIM, rm), jnp.float32),
            pltpu.SemaphoreType.DMA((_TC_NBUF,)),
            pltpu.SemaphoreType.DMA,
            pltpu.SemaphoreType.DMA,
        ],
    )(user_t, movie_t, w_mat)


_mesh = plsc.VectorSubcoreMesh(
    core_axis_name="c", subcore_axis_name="s", num_cores=NUM_CORES,
    num_subcores=NUM_SUBCORES)


@functools.partial(
    pl.kernel,
    out_type=jax.ShapeDtypeStruct((BATCH,), jnp.float32),
    mesh=_mesh,
    compiler_params=pltpu.CompilerParams(needs_layout_passes=False,
                                         use_tc_tiling_on_sc=False),
    scratch_types=[
        pltpu.VMEM((BPW,), jnp.int32),
        pltpu.VMEM((BPW,), jnp.int32),
        pltpu.VMEM((BPW,), jnp.float32),
        pltpu.VMEM((BPW,), jnp.float32),
        pltpu.VMEM((LANES,), jnp.float32),
        pltpu.VMEM((BPW,), jnp.float32),
        pltpu.SemaphoreType.DMA,
        pltpu.SemaphoreType.DMA,
    ],
)
def _gather_add(users_hbm, movies_hbm, uw_hbm, mw_hbm, b_hbm, out_hbm,
                uidx, midx, gu, gm, bvec, outv, sem_u, sem_m):
    wid = lax.axis_index("s") * NUM_CORES + lax.axis_index("c")
    base = wid * BPW
    pltpu.sync_copy(users_hbm.at[pl.ds(base, BPW)], uidx)
    pltpu.sync_copy(movies_hbm.at[pl.ds(base, BPW)], midx)
    pltpu.sync_copy(b_hbm, bvec)
    copies = []
    for c in range(NCHUNK):
        sl = pl.ds(c * CHUNK, CHUNK)
        copies.append(pltpu.async_copy(uw_hbm.at[uidx.at[sl]], gu.at[sl],
                                       sem_u))
        copies.append(pltpu.async_copy(mw_hbm.at[midx.at[sl]], gm.at[sl],
                                       sem_m))
    for cp in copies:
        cp.wait()
    b_val = bvec[...]
    for s in range(BPW // LANES):
        sl = pl.ds(s * LANES, LANES)
        outv[sl] = gu[sl] + gm[sl] + b_val
    pltpu.sync_copy(outv, out_hbm.at[pl.ds(base, BPW)])


def kernel(users, movies, user_table, movie_table, W, b):
    w = W.reshape(-1)
    w_mat = jnp.stack([w[:EMBED_DIM], w[EMBED_DIM:]], axis=1)
    uw, mw = _fused_colsum(user_table.T, movie_table.T, w_mat)
    bvec = jnp.broadcast_to(b.reshape(()), (LANES,))
    out = _gather_add(users.astype(jnp.int32), movies.astype(jnp.int32),
                      uw, mw, bvec)
    return out.reshape(BATCH, 1)
